# baseline (speedup 1.0000x reference)
import functools
import math

import jax
import jax.numpy as jnp
import numpy as np
from jax import lax
from jax.experimental import pallas as pl
from jax.experimental.pallas import tpu as pltpu
from jax.experimental.pallas import tpu_sc as plsc

NC = 2
NS = 16
NW = NC * NS
LANES = 16

_B, _S, _D = 4, 2048, 2048
_N = _B * _S

SC_ROWS = 6144
TC_ROWS = _N - SC_ROWS

ROWS_PER_W = SC_ROWS // NW
CHUNK = 8
NBUF = 3
NUM_CHUNKS = ROWS_PER_W // CHUNK

RB = 8

_KCOL = np.arange(_D, dtype=np.float64)
_DIV_ROW = np.exp(-(_KCOL - (_KCOL % 2)) * (math.log(10000.0) / _D))
_OFF_ROW = np.where(_KCOL % 2 == 1, np.pi / 2.0, 0.0)
_DIV8_NP = np.tile(_DIV_ROW.astype(np.float32)[None, :], (RB, 1))
_OFF8_NP = np.tile(_OFF_ROW.astype(np.float32)[None, :], (RB, 1))


def _sc_body(x_hbm, pos_hbm, pe_hbm, out_hbm, idx_v,
             rows0, rows1, rows2, xb0, xb1, xb2,
             g0, g1, g2, xs0, xs1, xs2, o0, o1, o2):
    rows = (rows0, rows1, rows2)
    xb = (xb0, xb1, xb2)
    gsem = (g0, g1, g2)
    xsem = (xs0, xs1, xs2)
    osem = (o0, o1, o2)

    wid = lax.axis_index("s") * NC + lax.axis_index("c")
    base = wid * ROWS_PER_W

    pltpu.sync_copy(pos_hbm.at[pl.ds(base, ROWS_PER_W)], idx_v)

    def gather_copy(c, b):
        return pltpu.make_async_copy(
            pe_hbm.at[idx_v.at[pl.ds(c * CHUNK, CHUNK)]], rows[b], gsem[b])

    def x_copy(c, b):
        return pltpu.make_async_copy(
            x_hbm.at[pl.ds(base + c * CHUNK, CHUNK)], xb[b], xsem[b])

    def out_copy(c, b):
        return pltpu.make_async_copy(
            xb[b], out_hbm.at[pl.ds(base + c * CHUNK, CHUNK)], osem[b])

    gather_copy(0, 0).start()
    x_copy(0, 0).start()

    def make_step(db):
        def step(cc):
            c = cc + db
            b = db
            nb = (db + 1) % NBUF

            @pl.when(c < NUM_CHUNKS)
            def _():
                @pl.when(c + 1 < NUM_CHUNKS)
                def _():
                    gather_copy(c + 1, nb).start()

                    @pl.when(c >= 2)
                    def _():
                        out_copy(c - 2, nb).wait()

                    x_copy(c + 1, nb).start()

                gather_copy(c, b).wait()
                x_copy(c, b).wait()

                for r in range(CHUNK):
                    def make_add(rr):
                        @plsc.parallel_loop(0, _D // LANES, unroll=8)
                        def _add(i):
                            sl = pl.ds(i * LANES, LANES)
                            plsc.addupdate(xb[b].at[rr, sl], rows[b][rr, sl])
                    make_add(r)

                out_copy(c, b).start()
        return step

    @pl.loop(0, NUM_CHUNKS + (-NUM_CHUNKS) % NBUF, step=NBUF)
    def outer(cc):
        for db in range(NBUF):
            make_step(db)(cc)

    for k in range(NUM_CHUNKS - 3, NUM_CHUNKS):
        out_copy(k, k % NBUF).wait()


def _sc_part(x2, pos, pe):
    body = functools.partial(
        pl.kernel,
        out_type=jax.ShapeDtypeStruct((_N, _D), jnp.float32),
        mesh=plsc.VectorSubcoreMesh(
            core_axis_name="c", subcore_axis_name="s",
            num_cores=NC, num_subcores=NS),
        scratch_types=(
            [pltpu.VMEM((ROWS_PER_W,), jnp.int32)]
            + [pltpu.VMEM((CHUNK, _D), jnp.float32)] * (2 * NBUF)
            + [pltpu.SemaphoreType.DMA] * (3 * NBUF)
        ),
    )(_sc_body)
    return body(x2, pos, pe)


def _tc_body(pos_ref, x_ref, div_ref, off_ref, out_ref):
    i = pl.program_id(0)
    ps = jnp.stack(
        [pos_ref[i * RB + j].astype(jnp.float32) for j in range(RB)])
    args = ps[:, None] * div_ref[...] + off_ref[...]
    out_ref[...] = x_ref[...] + jnp.sin(args)


def _tc_part(x2, pos, div8, off8):
    def x_map(i, pos_pref):
        return (i, 0)

    def const_map(i, pos_pref):
        return (0, 0)

    grid_spec = pltpu.PrefetchScalarGridSpec(
        num_scalar_prefetch=1,
        grid=(TC_ROWS // RB,),
        in_specs=[
            pl.BlockSpec((RB, _D), x_map),
            pl.BlockSpec((RB, _D), const_map),
            pl.BlockSpec((RB, _D), const_map),
        ],
        out_specs=pl.BlockSpec((RB, _D), x_map),
    )
    return pl.pallas_call(
        _tc_body,
        grid_spec=grid_spec,
        out_shape=jax.ShapeDtypeStruct((TC_ROWS, _D), jnp.float32),
    )(pos, x2, div8, off8)


@jax.jit
def _pe_add(x2, pos, pe):
    sc_full = _sc_part(x2, pos, pe)
    tc_out = _tc_part(x2[SC_ROWS:], pos[SC_ROWS:],
                      jnp.asarray(_DIV8_NP), jnp.asarray(_OFF8_NP))
    return lax.dynamic_update_slice(sc_full, tc_out, (SC_ROWS, 0))


def kernel(x, positions, pe):
    B, S, D = x.shape
    x2 = x.reshape(B * S, D)
    pos = positions.reshape(B * S)
    out = _pe_add(x2, pos, pe)
    return out.reshape(B, S, D)


# speedup vs baseline: 1.0005x; 1.0005x over previous
import functools
import math

import jax
import jax.numpy as jnp
import numpy as np
from jax import lax
from jax.experimental import pallas as pl
from jax.experimental.pallas import tpu as pltpu
from jax.experimental.pallas import tpu_sc as plsc

NC = 2
NS = 16
NW = NC * NS
LANES = 16

_B, _S, _D = 4, 2048, 2048
_N = _B * _S

SC_ROWS = 6144
TC_ROWS = _N - SC_ROWS

ROWS_PER_W = SC_ROWS // NW
CHUNK = 8
NBUF = 3
NUM_CHUNKS = ROWS_PER_W // CHUNK

RB = 8

_KCOL = np.arange(_D, dtype=np.float64)
_DIV_ROW = np.exp(-(_KCOL - (_KCOL % 2)) * (math.log(10000.0) / _D))
_OFF_ROW = np.where(_KCOL % 2 == 1, np.pi / 2.0, 0.0)
_DIV8_NP = np.tile(_DIV_ROW.astype(np.float32)[None, :], (RB, 1))
_OFF8_NP = np.tile(_OFF_ROW.astype(np.float32)[None, :], (RB, 1))


def _sc_body(x_hbm, pos_hbm, pe_hbm, out_hbm, idx_v,
             rows0, rows1, rows2, xb0, xb1, xb2,
             g0, g1, g2, xs0, xs1, xs2, o0, o1, o2):
    rows = (rows0, rows1, rows2)
    xb = (xb0, xb1, xb2)
    gsem = (g0, g1, g2)
    xsem = (xs0, xs1, xs2)
    osem = (o0, o1, o2)

    wid = lax.axis_index("s") * NC + lax.axis_index("c")
    base = wid * ROWS_PER_W

    pltpu.sync_copy(pos_hbm.at[pl.ds(base, ROWS_PER_W)], idx_v)

    def gather_copy(c, b):
        return pltpu.make_async_copy(
            pe_hbm.at[idx_v.at[pl.ds(c * CHUNK, CHUNK)]], rows[b], gsem[b])

    def x_copy(c, b):
        return pltpu.make_async_copy(
            x_hbm.at[pl.ds(base + c * CHUNK, CHUNK)], xb[b], xsem[b])

    def out_copy(c, b):
        return pltpu.make_async_copy(
            xb[b], out_hbm.at[pl.ds(base + c * CHUNK, CHUNK)], osem[b])

    gather_copy(0, 0).start()
    x_copy(0, 0).start()

    def make_step(db):
        def step(cc):
            c = cc + db
            b = db
            nb = (db + 1) % NBUF

            @pl.when(c < NUM_CHUNKS)
            def _():
                @pl.when(c + 1 < NUM_CHUNKS)
                def _():
                    gather_copy(c + 1, nb).start()

                    @pl.when(c >= 2)
                    def _():
                        out_copy(c - 2, nb).wait()

                    x_copy(c + 1, nb).start()

                gather_copy(c, b).wait()
                x_copy(c, b).wait()

                for r in range(CHUNK):
                    def make_add(rr):
                        @plsc.parallel_loop(0, _D // LANES, unroll=8)
                        def _add(i):
                            sl = pl.ds(i * LANES, LANES)
                            plsc.addupdate(xb[b].at[rr, sl], rows[b][rr, sl])
                    make_add(r)

                out_copy(c, b).start()
        return step

    @pl.loop(0, NUM_CHUNKS + (-NUM_CHUNKS) % NBUF, step=NBUF)
    def outer(cc):
        for db in range(NBUF):
            make_step(db)(cc)

    for k in range(NUM_CHUNKS - 3, NUM_CHUNKS):
        out_copy(k, k % NBUF).wait()


def _sc_part(x2, pos, pe):
    body = functools.partial(
        pl.kernel,
        out_type=jax.ShapeDtypeStruct((_N, _D), jnp.float32),
        mesh=plsc.VectorSubcoreMesh(
            core_axis_name="c", subcore_axis_name="s",
            num_cores=NC, num_subcores=NS),
        scratch_types=(
            [pltpu.VMEM((ROWS_PER_W,), jnp.int32)]
            + [pltpu.VMEM((CHUNK, _D), jnp.float32)] * (2 * NBUF)
            + [pltpu.SemaphoreType.DMA] * (3 * NBUF)
        ),
    )(_sc_body)
    return body(x2, pos, pe)


def _tc_body(pos_ref, x_ref, div_ref, off_ref, out_ref):
    i = pl.program_id(0)
    ps = jnp.stack(
        [pos_ref[i * RB + j].astype(jnp.float32) for j in range(RB)])
    args = ps[:, None] * div_ref[...] + off_ref[...]
    out_ref[...] = x_ref[...] + jnp.sin(args)


def _tc_part(x2, pos, div8, off8):
    def x_map(i, pos_pref):
        return (i, 0)

    def const_map(i, pos_pref):
        return (0, 0)

    grid_spec = pltpu.PrefetchScalarGridSpec(
        num_scalar_prefetch=1,
        grid=(TC_ROWS // RB,),
        in_specs=[
            pl.BlockSpec((RB, _D), x_map),
            pl.BlockSpec((RB, _D), const_map),
            pl.BlockSpec((RB, _D), const_map),
        ],
        out_specs=pl.BlockSpec((RB, _D), x_map),
    )
    return pl.pallas_call(
        _tc_body,
        grid_spec=grid_spec,
        out_shape=jax.ShapeDtypeStruct((TC_ROWS, _D), jnp.float32),
    )(pos, x2, div8, off8)


@jax.jit
def _pe_add(x2, pos, pe):
    tc_out = _tc_part(x2[SC_ROWS:], pos[SC_ROWS:],
                      jnp.asarray(_DIV8_NP), jnp.asarray(_OFF8_NP))
    sc_full = _sc_part(x2, pos, pe)
    return lax.dynamic_update_slice(sc_full, tc_out, (SC_ROWS, 0))


def kernel(x, positions, pe):
    B, S, D = x.shape
    x2 = x.reshape(B * S, D)
    pos = positions.reshape(B * S)
    out = _pe_add(x2, pos, pe)
    return out.reshape(B, S, D)


# speedup vs baseline: 2.4016x; 2.4003x over previous
import functools

import jax
import jax.numpy as jnp
from jax import lax
from jax.experimental import pallas as pl
from jax.experimental.pallas import tpu as pltpu
from jax.experimental.pallas import tpu_sc as plsc

NC = 2
NS = 16
NW = NC * NS
LANES = 16

_B, _S, _D = 4, 2048, 2048
_N = _B * _S
ROWS_PER_W = _N // NW
CHUNK = 8
NBUF = 3
NUM_CHUNKS = ROWS_PER_W // CHUNK


def _pe_add_body(x_hbm, pos_hbm, pe_hbm, out_hbm, idx_v,
                 rows0, rows1, rows2, xb0, xb1, xb2,
                 g0, g1, g2, xs0, xs1, xs2, o0, o1, o2):
    rows = (rows0, rows1, rows2)
    xb = (xb0, xb1, xb2)
    gsem = (g0, g1, g2)
    xsem = (xs0, xs1, xs2)
    osem = (o0, o1, o2)

    wid = lax.axis_index("s") * NC + lax.axis_index("c")
    base = wid * ROWS_PER_W

    pltpu.sync_copy(pos_hbm.at[pl.ds(base, ROWS_PER_W)], idx_v)

    def gather_copy(c, b):
        return pltpu.make_async_copy(
            pe_hbm.at[idx_v.at[pl.ds(c * CHUNK, CHUNK)]], rows[b], gsem[b])

    def x_copy(c, b):
        return pltpu.make_async_copy(
            x_hbm.at[pl.ds(base + c * CHUNK, CHUNK)], xb[b], xsem[b])

    def out_copy(c, b):
        return pltpu.make_async_copy(
            xb[b], out_hbm.at[pl.ds(base + c * CHUNK, CHUNK)], osem[b])

    gather_copy(0, 0).start()
    x_copy(0, 0).start()

    def make_step(db):
        def step(cc):
            c = cc + db
            b = db
            nb = (db + 1) % NBUF

            @pl.when(c < NUM_CHUNKS)
            def _():
                @pl.when(c + 1 < NUM_CHUNKS)
                def _():
                    gather_copy(c + 1, nb).start()

                    @pl.when(c >= 2)
                    def _():
                        out_copy(c - 2, nb).wait()

                    x_copy(c + 1, nb).start()

                gather_copy(c, b).wait()
                x_copy(c, b).wait()

                for r in range(CHUNK):
                    def make_add(rr):
                        @plsc.parallel_loop(0, _D // LANES, unroll=16)
                        def _add(i):
                            sl = pl.ds(i * LANES, LANES)
                            plsc.addupdate(xb[b].at[rr, sl], rows[b][rr, sl])
                    make_add(r)

                out_copy(c, b).start()
        return step

    @pl.loop(0, NUM_CHUNKS + (-NUM_CHUNKS) % NBUF, step=NBUF)
    def outer(cc):
        for db in range(NBUF):
            make_step(db)(cc)

    for k in range(NUM_CHUNKS - 3, NUM_CHUNKS):
        out_copy(k, k % NBUF).wait()


@jax.jit
def _pe_add(x2, pos, pe):
    body = functools.partial(
        pl.kernel,
        out_type=jax.ShapeDtypeStruct((_N, _D), jnp.float32),
        mesh=plsc.VectorSubcoreMesh(
            core_axis_name="c", subcore_axis_name="s",
            num_cores=NC, num_subcores=NS),
        scratch_types=(
            [pltpu.VMEM((ROWS_PER_W,), jnp.int32)]
            + [pltpu.VMEM((CHUNK, _D), jnp.float32)] * (2 * NBUF)
            + [pltpu.SemaphoreType.DMA] * (3 * NBUF)
        ),
    )(_pe_add_body)
    return body(x2, pos, pe)


def kernel(x, positions, pe):
    B, S, D = x.shape
    x2 = x.reshape(B * S, D)
    pos = positions.reshape(B * S)
    out = _pe_add(x2, pos, pe)
    return out.reshape(B, S, D)


# speedup vs baseline: 2.4716x; 1.0292x over previous
import functools

import jax
import jax.numpy as jnp
from jax import lax
from jax.experimental import pallas as pl
from jax.experimental.pallas import tpu as pltpu
from jax.experimental.pallas import tpu_sc as plsc

NC = 2
NS = 16
NW = NC * NS
LANES = 16

_B, _S, _D = 4, 2048, 2048
_N = _B * _S
ROWS_PER_W = _N // NW
CHUNK = 8
NBUF = 3
NUM_CHUNKS = ROWS_PER_W // CHUNK


def _pe_add_body(x_hbm, pos_hbm, pe_hbm, out_hbm, idx_v,
                 rows0, rows1, rows2, xb0, xb1, xb2,
                 g0, g1, g2, xs0, xs1, xs2, o0, o1, o2):
    rows = (rows0, rows1, rows2)
    xb = (xb0, xb1, xb2)
    gsem = (g0, g1, g2)
    xsem = (xs0, xs1, xs2)
    osem = (o0, o1, o2)

    wid = lax.axis_index("s") * NC + lax.axis_index("c")
    base = wid * ROWS_PER_W

    pltpu.sync_copy(pos_hbm.at[pl.ds(base, ROWS_PER_W)], idx_v)

    def gather_copy(c, b):
        return pltpu.make_async_copy(
            pe_hbm.at[idx_v.at[pl.ds(c * CHUNK, CHUNK)]], rows[b], gsem[b])

    def x_copy(c, b):
        return pltpu.make_async_copy(
            x_hbm.at[pl.ds(base + c * CHUNK, CHUNK)], xb[b], xsem[b])

    def out_copy(c, b):
        return pltpu.make_async_copy(
            xb[b], out_hbm.at[pl.ds(base + c * CHUNK, CHUNK)], osem[b])

    gather_copy(0, 0).start()
    gather_copy(1, 1).start()
    x_copy(0, 0).start()

    def make_step(db):
        def step(cc):
            c = cc + db
            b = db
            nb = (db + 1) % NBUF

            nb2 = (db + 2) % NBUF

            @pl.when(c < NUM_CHUNKS)
            def _():
                @pl.when(c + 2 < NUM_CHUNKS)
                def _():
                    gather_copy(c + 2, nb2).start()

                @pl.when(c + 1 < NUM_CHUNKS)
                def _():
                    @pl.when(c >= 2)
                    def _():
                        out_copy(c - 2, nb).wait()

                    x_copy(c + 1, nb).start()

                gather_copy(c, b).wait()
                x_copy(c, b).wait()

                for r in range(CHUNK):
                    def make_add(rr):
                        @plsc.parallel_loop(0, _D // LANES, unroll=8)
                        def _add(i):
                            sl = pl.ds(i * LANES, LANES)
                            plsc.addupdate(xb[b].at[rr, sl], rows[b][rr, sl])
                    make_add(r)

                out_copy(c, b).start()
        return step

    @pl.loop(0, NUM_CHUNKS + (-NUM_CHUNKS) % NBUF, step=NBUF)
    def outer(cc):
        for db in range(NBUF):
            make_step(db)(cc)

    for k in range(NUM_CHUNKS - 3, NUM_CHUNKS):
        out_copy(k, k % NBUF).wait()


@jax.jit
def _pe_add(x2, pos, pe):
    body = functools.partial(
        pl.kernel,
        out_type=jax.ShapeDtypeStruct((_N, _D), jnp.float32),
        mesh=plsc.VectorSubcoreMesh(
            core_axis_name="c", subcore_axis_name="s",
            num_cores=NC, num_subcores=NS),
        scratch_types=(
            [pltpu.VMEM((ROWS_PER_W,), jnp.int32)]
            + [pltpu.VMEM((CHUNK, _D), jnp.float32)] * (2 * NBUF)
            + [pltpu.SemaphoreType.DMA] * (3 * NBUF)
        ),
    )(_pe_add_body)
    return body(x2, pos, pe)


def kernel(x, positions, pe):
    B, S, D = x.shape
    x2 = x.reshape(B * S, D)
    pos = positions.reshape(B * S)
    out = _pe_add(x2, pos, pe)
    return out.reshape(B, S, D)
